# baseline (speedup 1.0000x reference)
import functools

import jax
import jax.numpy as jnp
from jax import lax
from jax.experimental import pallas as pl
from jax.experimental.pallas import tpu as pltpu
from jax.experimental.pallas import tpu_sc as plsc

_N = 4096
_DF = 1024
_DC = 512
_DH = 256
_NCLS = 2
_K = 8
_BLK = 512
_NB = _N // _BLK

_NEG = -3.0e38
_POS = 3.0e38
_IMAX = 2**31 - 1


def _tc_body(x_ref, wfc_ref, bfc_ref, wa1_ref, ba1_ref, wa2_ref, ba2_ref,
             wa3_ref, ba3_ref, wbag_ref, bbag_ref,
             h_ref, attlin_ref, alin_ref, score_ref, prob_ref,
             yhat_ref, pred_ref,
             atts_ref, macc_ref, m_ref, z_ref):
    i = pl.program_id(0)

    @pl.when(i == 0)
    def _init():
        m_ref[0, 0] = jnp.float32(-jnp.inf)
        z_ref[0, 0] = jnp.float32(0.0)
        macc_ref[...] = jnp.zeros_like(macc_ref)

    x = x_ref[...]
    h = jnp.maximum(
        jnp.dot(x, wfc_ref[...], preferred_element_type=jnp.float32)
        + bfc_ref[...], 0.0)
    h_ref[...] = h
    a1 = jnp.tanh(
        jnp.dot(x, wa1_ref[...], preferred_element_type=jnp.float32)
        + ba1_ref[...])
    a2 = jax.nn.sigmoid(
        jnp.dot(x, wa2_ref[...], preferred_element_type=jnp.float32)
        + ba2_ref[...])
    s = jnp.dot(a1 * a2, wa3_ref[...],
                preferred_element_type=jnp.float32) + ba3_ref[0, 0]
    atts_ref[pl.ds(i * _BLK, _BLK), :] = s

    m_old = m_ref[0, 0]
    m_new = jnp.maximum(m_old, jnp.max(s))
    scale = jnp.exp(m_old - m_new)
    e = jnp.exp(s - m_new)
    z_ref[0, 0] = z_ref[0, 0] * scale + jnp.sum(e)
    contrib = lax.dot_general(e, h, (((0,), (0,)), ((), ())),
                              preferred_element_type=jnp.float32)
    macc_ref[...] = macc_ref[...] * scale + contrib
    m_ref[0, 0] = m_new

    @pl.when(i == _NB - 1)
    def _fin():
        m = m_ref[0, 0]
        z = z_ref[0, 0]
        atts = atts_ref[...].reshape(_N // 128, 128)
        attlin_ref[...] = atts
        alin_ref[...] = jnp.exp(atts - m) / z
        big_m = macc_ref[...] / z
        sc = jnp.dot(big_m, wbag_ref[...],
                     preferred_element_type=jnp.float32) + bbag_ref[...]
        score_ref[...] = sc
        ee = jnp.exp(sc - jnp.max(sc))
        p = ee / jnp.sum(ee)
        prob_ref[...] = p
        yhat_ref[...] = jnp.where(sc[0, 1] > sc[0, 0], 1, 0).astype(
            jnp.int32).reshape(1, 1)
        pred_ref[...] = jnp.where(p[0, 1] > p[0, 0], 1, 0).astype(
            jnp.int32).reshape(1, 1)


def _tc_call(x, wfc, bfc, wa1, ba1, wa2, ba2, wa3, ba3, wbag, bbag):
    f32 = jnp.float32
    full = lambda shape: pl.BlockSpec(shape, lambda i: (0, 0))
    return pl.pallas_call(
        _tc_body,
        grid=(_NB,),
        in_specs=[
            pl.BlockSpec((_BLK, _DF), lambda i: (i, 0)),
            full((_DF, _DC)), full((1, _DC)),
            full((_DF, _DH)), full((1, _DH)),
            full((_DF, _DH)), full((1, _DH)),
            full((_DH, 1)), full((1, 1)),
            full((_DC, _NCLS)), full((1, _NCLS)),
        ],
        out_specs=[
            pl.BlockSpec((_BLK, _DC), lambda i: (i, 0)),
            full((_N // 128, 128)),
            full((_N // 128, 128)),
            full((1, _NCLS)), full((1, _NCLS)),
            full((1, 1)), full((1, 1)),
        ],
        out_shape=[
            jax.ShapeDtypeStruct((_N, _DC), f32),
            jax.ShapeDtypeStruct((_N // 128, 128), f32),
            jax.ShapeDtypeStruct((_N // 128, 128), f32),
            jax.ShapeDtypeStruct((1, _NCLS), f32),
            jax.ShapeDtypeStruct((1, _NCLS), f32),
            jax.ShapeDtypeStruct((1, 1), jnp.int32),
            jax.ShapeDtypeStruct((1, 1), jnp.int32),
        ],
        scratch_shapes=[
            pltpu.VMEM((_N, 1), f32),
            pltpu.VMEM((1, _DC), f32),
            pltpu.SMEM((1, 1), f32),
            pltpu.SMEM((1, 1), f32),
        ],
        compiler_params=pltpu.CompilerParams(
            dimension_semantics=("arbitrary",)),
    )(x, wfc, bfc, wa1, ba1, wa2, ba2, wa3, ba3, wbag, bbag)


_NW = 16
_CH = _N // _NW
_NCH = _CH // 16


def _select_pass(score_ref, ids_base, n_vregs, is_top, lane_off, k,
                 tvals, tids):
    lanes = lax.iota(jnp.int32, 16)
    fill = _NEG if is_top else _POS

    def body(t, carry):
        tvals, tids = carry
        vs = [score_ref[pl.ds(j * 16, 16)] for j in range(n_vregs)]
        m0 = vs[0]
        for j in range(1, n_vregs):
            m0 = jnp.maximum(m0, vs[j]) if is_top else jnp.minimum(m0, vs[j])
        best = jnp.max(m0) if is_top else jnp.min(m0)

        mi = jnp.full((16,), _IMAX, jnp.int32)
        for j in range(n_vregs):
            idv = lanes + ids_base + j * 16
            mi = jnp.minimum(mi, jnp.where(vs[j] == best, idv, _IMAX))
        chosen = jnp.min(mi)

        tvals = jnp.where(lanes == (t + lane_off), best, tvals)
        tids = jnp.where(lanes == (t + lane_off), chosen, tids)

        for j in range(n_vregs):
            idv = lanes + ids_base + j * 16
            score_ref[pl.ds(j * 16, 16)] = jnp.where(idv == chosen, fill,
                                                     vs[j])
        return (tvals, tids)

    return lax.fori_loop(0, k, body, (tvals, tids))


def _merge_pass(work_ref, cid_ref, is_top, lane_off, gvals, gids):
    lanes = lax.iota(jnp.int32, 16)
    fill = _NEG if is_top else _POS

    def body(t, carry):
        gvals, gids = carry
        vs = [work_ref[pl.ds(j * 16, 16)] for j in range(_NW)]
        ivs = [cid_ref[pl.ds(j * 16, 16)] for j in range(_NW)]
        m0 = vs[0]
        for j in range(1, _NW):
            m0 = jnp.maximum(m0, vs[j]) if is_top else jnp.minimum(m0, vs[j])
        best = jnp.max(m0) if is_top else jnp.min(m0)

        mi = jnp.full((16,), _IMAX, jnp.int32)
        for j in range(_NW):
            mi = jnp.minimum(mi, jnp.where(vs[j] == best, ivs[j], _IMAX))
        chosen = jnp.min(mi)

        gvals = jnp.where(lanes == (t + lane_off), best, gvals)
        gids = jnp.where(lanes == (t + lane_off), chosen, gids)

        for j in range(_NW):
            work_ref[pl.ds(j * 16, 16)] = jnp.where(ivs[j] == chosen, fill,
                                                    vs[j])
        return (gvals, gids)

    return lax.fori_loop(0, _K, body, (gvals, gids))


def _sc_body(att_hbm, stagev_hbm, stagei_hbm, ids_hbm,
             loc_ref, loc2_ref, sv_ref, si_ref,
             cv_ref, ci_ref, work_ref, cid_ref):
    c = lax.axis_index("c")
    s = lax.axis_index("s")
    lanes = lax.iota(jnp.int32, 16)
    zf = jnp.zeros((16,), jnp.float32)
    zi = jnp.zeros((16,), jnp.int32)

    @pl.when(c == 0)
    def _phase1():
        base = s * _CH
        pltpu.sync_copy(att_hbm.at[pl.ds(base, _CH)], loc_ref)
        for j in range(_NCH):
            loc2_ref[pl.ds(j * 16, 16)] = loc_ref[pl.ds(j * 16, 16)]
        tvals, tids = _select_pass(loc_ref, base, _NCH, True, 0, _K, zf, zi)
        tvals, tids = _select_pass(loc2_ref, base, _NCH, False, _K, _K,
                                   tvals, tids)
        sv_ref[...] = tvals
        si_ref[...] = tids
        pltpu.sync_copy(sv_ref, stagev_hbm.at[pl.ds(s * 16, 16)])
        pltpu.sync_copy(si_ref, stagei_hbm.at[pl.ds(s * 16, 16)])

    plsc.subcore_barrier()

    @pl.when(jnp.logical_and(c == 0, s == 0))
    def _merge():
        pltpu.sync_copy(stagev_hbm, cv_ref)
        pltpu.sync_copy(stagei_hbm, ci_ref)

        for j in range(_NW):
            cv = cv_ref[pl.ds(j * 16, 16)]
            ci = ci_ref[pl.ds(j * 16, 16)]
            work_ref[pl.ds(j * 16, 16)] = jnp.where(lanes < _K, cv, _NEG)
            cid_ref[pl.ds(j * 16, 16)] = jnp.where(lanes < _K, ci, _IMAX)
        gvals, gids = _merge_pass(work_ref, cid_ref, True, 0, zf, zi)

        for j in range(_NW):
            cv = cv_ref[pl.ds(j * 16, 16)]
            ci = ci_ref[pl.ds(j * 16, 16)]
            work_ref[pl.ds(j * 16, 16)] = jnp.where(lanes >= _K, cv, _POS)
            cid_ref[pl.ds(j * 16, 16)] = jnp.where(lanes >= _K, ci, _IMAX)
        gvals, gids = _merge_pass(work_ref, cid_ref, False, _K, gvals, gids)

        si_ref[...] = jnp.clip(gids, 0, _N - 1)
        pltpu.sync_copy(si_ref, ids_hbm)


def _sc_call(att):
    f32 = jnp.float32
    i32 = jnp.int32
    mesh = plsc.VectorSubcoreMesh(core_axis_name="c", subcore_axis_name="s")
    fn = pl.kernel(
        _sc_body,
        out_type=[
            jax.ShapeDtypeStruct((_NW * 16,), f32),
            jax.ShapeDtypeStruct((_NW * 16,), i32),
            jax.ShapeDtypeStruct((16,), i32),
        ],
        mesh=mesh,
        scratch_types=[
            pltpu.VMEM((_CH,), f32),
            pltpu.VMEM((_CH,), f32),
            pltpu.VMEM((16,), f32),
            pltpu.VMEM((16,), i32),
            pltpu.VMEM((_NW * 16,), f32),
            pltpu.VMEM((_NW * 16,), i32),
            pltpu.VMEM((_NW * 16,), f32),
            pltpu.VMEM((_NW * 16,), i32),
        ],
        compiler_params=pltpu.CompilerParams(needs_layout_passes=False),
    )
    _, _, ids = fn(att)
    return ids


def _tc2_body(ids_smem, hrow_ref, wins_ref, bins_ref, unnorm_ref, prob_ref,
              acc_ref):
    i = pl.program_id(0)
    r = ids_smem[i] % 8
    acc_ref[pl.ds(i, 1), :] = hrow_ref[pl.ds(r, 1), :]

    @pl.when(i == 2 * _K - 1)
    def _fin():
        logits = jnp.dot(acc_ref[...], wins_ref[...],
                         preferred_element_type=jnp.float32) + bins_ref[...]
        unnorm_ref[...] = logits
        mm = jnp.max(logits, axis=1, keepdims=True)
        e = jnp.exp(logits - mm)
        prob_ref[...] = e / jnp.sum(e, axis=1, keepdims=True)


def _tc2_call(ids, h, wins, bins):
    f32 = jnp.float32
    grid_spec = pltpu.PrefetchScalarGridSpec(
        num_scalar_prefetch=1,
        grid=(2 * _K,),
        in_specs=[
            pl.BlockSpec((8, _DC), lambda i, ids_ref: (ids_ref[i] // 8, 0)),
            pl.BlockSpec((_DC, _NCLS), lambda i, ids_ref: (0, 0)),
            pl.BlockSpec((1, _NCLS), lambda i, ids_ref: (0, 0)),
        ],
        out_specs=[
            pl.BlockSpec((2 * _K, _NCLS), lambda i, ids_ref: (0, 0)),
            pl.BlockSpec((2 * _K, _NCLS), lambda i, ids_ref: (0, 0)),
        ],
        scratch_shapes=[pltpu.VMEM((2 * _K, _DC), f32)],
    )
    return pl.pallas_call(
        _tc2_body,
        grid_spec=grid_spec,
        out_shape=[
            jax.ShapeDtypeStruct((2 * _K, _NCLS), f32),
            jax.ShapeDtypeStruct((2 * _K, _NCLS), f32),
        ],
        compiler_params=pltpu.CompilerParams(
            dimension_semantics=("arbitrary",)),
    )(ids, h, wins, bins)


def kernel(img_features, slide_label, W_fc, b_fc, W_a1, b_a1, W_a2, b_a2,
           W_a3, b_a3, W_ins, b_ins, W_bag, b_bag):
    x = img_features.reshape(_N, _DF)
    (h, attlin, alin, score, prob, yhat, pred) = _tc_call(
        x, W_fc, b_fc.reshape(1, _DC), W_a1, b_a1.reshape(1, _DH),
        W_a2, b_a2.reshape(1, _DH), W_a3, b_a3.reshape(1, 1),
        W_bag, b_bag.reshape(1, _NCLS))

    ids = _sc_call(attlin.reshape(_N))
    unnorm, ins_prob = _tc2_call(ids, h, W_ins, b_ins.reshape(1, _NCLS))

    ins_labels = jnp.concatenate(
        [jnp.ones((_K,), jnp.int32), jnp.zeros((_K,), jnp.int32)], axis=0)
    y_true = jax.nn.one_hot(jnp.asarray(slide_label), _NCLS)
    return (attlin.reshape(_N, 1), alin.reshape(_N, 1), h, ins_labels,
            unnorm, ins_prob, score, prob,
            yhat.reshape(1), y_true, pred.reshape(1))


# speedup vs baseline: 1.2996x; 1.2996x over previous
import functools

import jax
import jax.numpy as jnp
from jax import lax
from jax.experimental import pallas as pl
from jax.experimental.pallas import tpu as pltpu
from jax.experimental.pallas import tpu_sc as plsc

_N = 4096
_DF = 1024
_DC = 512
_DH = 256
_NCLS = 2
_K = 8
_BLK = 512
_NB = _N // _BLK

_NEG = -3.0e38
_POS = 3.0e38
_IMAX = 2**31 - 1


def _tc_body(x_ref, wfc_ref, bfc_ref, wa1_ref, ba1_ref, wa2_ref, ba2_ref,
             wa3_ref, ba3_ref, wbag_ref, bbag_ref,
             h_ref, attlin_ref, alin_ref, score_ref, prob_ref,
             yhat_ref, pred_ref,
             atts_ref, macc_ref, m_ref, z_ref):
    i = pl.program_id(0)

    @pl.when(i == 0)
    def _init():
        m_ref[0, 0] = jnp.float32(-jnp.inf)
        z_ref[0, 0] = jnp.float32(0.0)
        macc_ref[...] = jnp.zeros_like(macc_ref)

    x = x_ref[...].reshape(_BLK, _DF)
    hs = jnp.dot(x, wfc_ref[...], preferred_element_type=jnp.float32)
    a1s = jnp.dot(x, wa1_ref[...], preferred_element_type=jnp.float32)
    a2s = jnp.dot(x, wa2_ref[...], preferred_element_type=jnp.float32)
    h = jnp.maximum(hs + bfc_ref[...], 0.0)
    h_ref[...] = h
    a1 = jnp.tanh(a1s + ba1_ref[...])
    a2 = jax.nn.sigmoid(a2s + ba2_ref[...])
    s = jnp.dot(a1 * a2, wa3_ref[...],
                preferred_element_type=jnp.float32) + ba3_ref[0, 0]
    atts_ref[pl.ds(i * _BLK, _BLK), :] = s

    m_old = m_ref[0, 0]
    m_new = jnp.maximum(m_old, jnp.max(s))
    scale = jnp.exp(m_old - m_new)
    e = jnp.exp(s - m_new)
    z_ref[0, 0] = z_ref[0, 0] * scale + jnp.sum(e)
    contrib = lax.dot_general(e, h, (((0,), (0,)), ((), ())),
                              preferred_element_type=jnp.float32)
    macc_ref[...] = macc_ref[...] * scale + contrib
    m_ref[0, 0] = m_new

    @pl.when(i == _NB - 1)
    def _fin():
        m = m_ref[0, 0]
        z = z_ref[0, 0]
        atts = atts_ref[...].reshape(_N // 128, 128)
        attlin_ref[...] = atts
        alin_ref[...] = jnp.exp(atts - m) / z
        big_m = macc_ref[...] / z
        sc = jnp.dot(big_m, wbag_ref[...],
                     preferred_element_type=jnp.float32) + bbag_ref[...]
        score_ref[...] = sc
        ee = jnp.exp(sc - jnp.max(sc))
        p = ee / jnp.sum(ee)
        prob_ref[...] = p
        yhat_ref[...] = jnp.where(sc[0, 1] > sc[0, 0], 1, 0).astype(
            jnp.int32).reshape(1, 1)
        pred_ref[...] = jnp.where(p[0, 1] > p[0, 0], 1, 0).astype(
            jnp.int32).reshape(1, 1)


def _tc_call(x, wfc, bfc, wa1, ba1, wa2, ba2, wa3, ba3, wbag, bbag):
    f32 = jnp.float32
    full = lambda shape: pl.BlockSpec(shape, lambda i: (0, 0))
    return pl.pallas_call(
        _tc_body,
        grid=(_NB,),
        in_specs=[
            pl.BlockSpec((_BLK, 8, 128), lambda i: (i, 0, 0)),
            full((_DF, _DC)), full((1, _DC)),
            full((_DF, _DH)), full((1, _DH)),
            full((_DF, _DH)), full((1, _DH)),
            full((_DH, 1)), full((1, 1)),
            full((_DC, _NCLS)), full((1, _NCLS)),
        ],
        out_specs=[
            pl.BlockSpec((_BLK, _DC), lambda i: (i, 0)),
            full((_N // 128, 128)),
            full((_N // 128, 128)),
            full((1, _NCLS)), full((1, _NCLS)),
            full((1, 1)), full((1, 1)),
        ],
        out_shape=[
            jax.ShapeDtypeStruct((_N, _DC), f32),
            jax.ShapeDtypeStruct((_N // 128, 128), f32),
            jax.ShapeDtypeStruct((_N // 128, 128), f32),
            jax.ShapeDtypeStruct((1, _NCLS), f32),
            jax.ShapeDtypeStruct((1, _NCLS), f32),
            jax.ShapeDtypeStruct((1, 1), jnp.int32),
            jax.ShapeDtypeStruct((1, 1), jnp.int32),
        ],
        scratch_shapes=[
            pltpu.VMEM((_N, 1), f32),
            pltpu.VMEM((1, _DC), f32),
            pltpu.SMEM((1, 1), f32),
            pltpu.SMEM((1, 1), f32),
        ],
        compiler_params=pltpu.CompilerParams(
            dimension_semantics=("arbitrary",)),
    )(x, wfc, bfc, wa1, ba1, wa2, ba2, wa3, ba3, wbag, bbag)


_NW = 16
_CH = _N // _NW
_NCH = _CH // 16


def _select_pass(score_ref, ids_base, n_vregs, is_top, lane_off, k,
                 tvals, tids):
    lanes = lax.iota(jnp.int32, 16)
    fill = _NEG if is_top else _POS

    def body(t, carry):
        tvals, tids = carry
        vs = [score_ref[pl.ds(j * 16, 16)] for j in range(n_vregs)]
        m0 = vs[0]
        for j in range(1, n_vregs):
            m0 = jnp.maximum(m0, vs[j]) if is_top else jnp.minimum(m0, vs[j])
        best = jnp.max(m0) if is_top else jnp.min(m0)

        mi = jnp.full((16,), _IMAX, jnp.int32)
        for j in range(n_vregs):
            idv = lanes + ids_base + j * 16
            mi = jnp.minimum(mi, jnp.where(vs[j] == best, idv, _IMAX))
        chosen = jnp.min(mi)

        tvals = jnp.where(lanes == (t + lane_off), best, tvals)
        tids = jnp.where(lanes == (t + lane_off), chosen, tids)

        for j in range(n_vregs):
            idv = lanes + ids_base + j * 16
            score_ref[pl.ds(j * 16, 16)] = jnp.where(idv == chosen, fill,
                                                     vs[j])
        return (tvals, tids)

    return lax.fori_loop(0, k, body, (tvals, tids))


def _merge_pass(work_ref, cid_ref, is_top, lane_off, gvals, gids):
    lanes = lax.iota(jnp.int32, 16)
    fill = _NEG if is_top else _POS

    def body(t, carry):
        gvals, gids = carry
        vs = [work_ref[pl.ds(j * 16, 16)] for j in range(_NW)]
        ivs = [cid_ref[pl.ds(j * 16, 16)] for j in range(_NW)]
        m0 = vs[0]
        for j in range(1, _NW):
            m0 = jnp.maximum(m0, vs[j]) if is_top else jnp.minimum(m0, vs[j])
        best = jnp.max(m0) if is_top else jnp.min(m0)

        mi = jnp.full((16,), _IMAX, jnp.int32)
        for j in range(_NW):
            mi = jnp.minimum(mi, jnp.where(vs[j] == best, ivs[j], _IMAX))
        chosen = jnp.min(mi)

        gvals = jnp.where(lanes == (t + lane_off), best, gvals)
        gids = jnp.where(lanes == (t + lane_off), chosen, gids)

        for j in range(_NW):
            work_ref[pl.ds(j * 16, 16)] = jnp.where(ivs[j] == chosen, fill,
                                                    vs[j])
        return (gvals, gids)

    return lax.fori_loop(0, _K, body, (gvals, gids))


def _sc_body(att_hbm, stagev_hbm, stagei_hbm, ids_hbm,
             loc_ref, loc2_ref, sv_ref, si_ref,
             cv_ref, ci_ref, work_ref, cid_ref):
    c = lax.axis_index("c")
    s = lax.axis_index("s")
    lanes = lax.iota(jnp.int32, 16)
    zf = jnp.zeros((16,), jnp.float32)
    zi = jnp.zeros((16,), jnp.int32)

    @pl.when(c == 0)
    def _phase1():
        base = s * _CH
        pltpu.sync_copy(att_hbm.at[pl.ds(base, _CH)], loc_ref)
        for j in range(_NCH):
            loc2_ref[pl.ds(j * 16, 16)] = loc_ref[pl.ds(j * 16, 16)]
        tvals, tids = _select_pass(loc_ref, base, _NCH, True, 0, _K, zf, zi)
        tvals, tids = _select_pass(loc2_ref, base, _NCH, False, _K, _K,
                                   tvals, tids)
        sv_ref[...] = tvals
        si_ref[...] = tids
        pltpu.sync_copy(sv_ref, stagev_hbm.at[pl.ds(s * 16, 16)])
        pltpu.sync_copy(si_ref, stagei_hbm.at[pl.ds(s * 16, 16)])

    plsc.subcore_barrier()

    @pl.when(jnp.logical_and(c == 0, s == 0))
    def _merge():
        pltpu.sync_copy(stagev_hbm, cv_ref)
        pltpu.sync_copy(stagei_hbm, ci_ref)

        for j in range(_NW):
            cv = cv_ref[pl.ds(j * 16, 16)]
            ci = ci_ref[pl.ds(j * 16, 16)]
            work_ref[pl.ds(j * 16, 16)] = jnp.where(lanes < _K, cv, _NEG)
            cid_ref[pl.ds(j * 16, 16)] = jnp.where(lanes < _K, ci, _IMAX)
        gvals, gids = _merge_pass(work_ref, cid_ref, True, 0, zf, zi)

        for j in range(_NW):
            cv = cv_ref[pl.ds(j * 16, 16)]
            ci = ci_ref[pl.ds(j * 16, 16)]
            work_ref[pl.ds(j * 16, 16)] = jnp.where(lanes >= _K, cv, _POS)
            cid_ref[pl.ds(j * 16, 16)] = jnp.where(lanes >= _K, ci, _IMAX)
        gvals, gids = _merge_pass(work_ref, cid_ref, False, _K, gvals, gids)

        si_ref[...] = jnp.clip(gids, 0, _N - 1)
        pltpu.sync_copy(si_ref, ids_hbm)


def _sc_call(att):
    f32 = jnp.float32
    i32 = jnp.int32
    mesh = plsc.VectorSubcoreMesh(core_axis_name="c", subcore_axis_name="s")
    fn = pl.kernel(
        _sc_body,
        out_type=[
            jax.ShapeDtypeStruct((_NW * 16,), f32),
            jax.ShapeDtypeStruct((_NW * 16,), i32),
            jax.ShapeDtypeStruct((16,), i32),
        ],
        mesh=mesh,
        scratch_types=[
            pltpu.VMEM((_CH,), f32),
            pltpu.VMEM((_CH,), f32),
            pltpu.VMEM((16,), f32),
            pltpu.VMEM((16,), i32),
            pltpu.VMEM((_NW * 16,), f32),
            pltpu.VMEM((_NW * 16,), i32),
            pltpu.VMEM((_NW * 16,), f32),
            pltpu.VMEM((_NW * 16,), i32),
        ],
        compiler_params=pltpu.CompilerParams(needs_layout_passes=False),
    )
    _, _, ids = fn(att)
    return ids


_GB = 1024
_NGB = _N // _GB


def _tc2_body(ids_ref, h_ref, wins_ref, bins_ref, unnorm_ref, prob_ref,
              acc_ref):
    i = pl.program_id(0)

    @pl.when(i == 0)
    def _init():
        acc_ref[...] = jnp.zeros_like(acc_ref)

    ids = ids_ref[...].reshape(2 * _K, 1)
    col = lax.broadcasted_iota(jnp.int32, (2 * _K, _GB), 1) + i * _GB
    onehot = (col == ids).astype(jnp.float32)
    acc_ref[...] += jnp.dot(onehot, h_ref[...],
                            preferred_element_type=jnp.float32)

    @pl.when(i == _NGB - 1)
    def _fin():
        logits = jnp.dot(acc_ref[...], wins_ref[...],
                         preferred_element_type=jnp.float32) + bins_ref[...]
        unnorm_ref[...] = logits
        mm = jnp.max(logits, axis=1, keepdims=True)
        e = jnp.exp(logits - mm)
        prob_ref[...] = e / jnp.sum(e, axis=1, keepdims=True)


def _tc2_call(ids, h, wins, bins):
    f32 = jnp.float32
    full = lambda shape: pl.BlockSpec(shape, lambda i: (0, 0))
    return pl.pallas_call(
        _tc2_body,
        grid=(_NGB,),
        in_specs=[
            full((1, 2 * _K)),
            pl.BlockSpec((_GB, _DC), lambda i: (i, 0)),
            full((_DC, _NCLS)),
            full((1, _NCLS)),
        ],
        out_specs=[full((2 * _K, _NCLS)), full((2 * _K, _NCLS))],
        out_shape=[
            jax.ShapeDtypeStruct((2 * _K, _NCLS), f32),
            jax.ShapeDtypeStruct((2 * _K, _NCLS), f32),
        ],
        scratch_shapes=[pltpu.VMEM((2 * _K, _DC), f32)],
        compiler_params=pltpu.CompilerParams(
            dimension_semantics=("arbitrary",)),
    )(ids, h, wins, bins)


def kernel(img_features, slide_label, W_fc, b_fc, W_a1, b_a1, W_a2, b_a2,
           W_a3, b_a3, W_ins, b_ins, W_bag, b_bag):
    x = img_features.reshape(_N, 8, 128)
    (h, attlin, alin, score, prob, yhat, pred) = _tc_call(
        x, W_fc, b_fc.reshape(1, _DC), W_a1, b_a1.reshape(1, _DH),
        W_a2, b_a2.reshape(1, _DH), W_a3, b_a3.reshape(1, 1),
        W_bag, b_bag.reshape(1, _NCLS))

    ids = _sc_call(attlin.reshape(_N))
    unnorm, ins_prob = _tc2_call(ids.reshape(1, 2 * _K), h, W_ins,
                                 b_ins.reshape(1, _NCLS))

    ins_labels = jnp.concatenate(
        [jnp.ones((_K,), jnp.int32), jnp.zeros((_K,), jnp.int32)], axis=0)
    y_true = jax.nn.one_hot(jnp.asarray(slide_label), _NCLS)
    return (attlin.reshape(_N, 1), alin.reshape(_N, 1), h, ins_labels,
            unnorm, ins_prob, score, prob,
            yhat.reshape(1), y_true, pred.reshape(1))


# speedup vs baseline: 1.3581x; 1.0450x over previous
import functools

import jax
import jax.numpy as jnp
from jax import lax
from jax.experimental import pallas as pl
from jax.experimental.pallas import tpu as pltpu
from jax.experimental.pallas import tpu_sc as plsc

_N = 4096
_DF = 1024
_DC = 512
_DH = 256
_NCLS = 2
_K = 8
_BLK = 512
_NB = _N // _BLK

_NEG = -3.0e38
_POS = 3.0e38
_IMAX = 2**31 - 1


def _tc_body(x_ref, wfc_ref, bfc_ref, wa1_ref, ba1_ref, wa2_ref, ba2_ref,
             wa3_ref, ba3_ref, wbag_ref, bbag_ref, wins_ref, bins_ref,
             h_ref, attlin_ref, alin_ref, lg0_ref, lg1_ref,
             score_ref, prob_ref, yhat_ref, pred_ref,
             atts_ref, macc_ref, lg_ref, m_ref, z_ref):
    i = pl.program_id(0)

    @pl.when(i == 0)
    def _init():
        m_ref[0, 0] = jnp.float32(-jnp.inf)
        z_ref[0, 0] = jnp.float32(0.0)
        macc_ref[...] = jnp.zeros_like(macc_ref)

    x = x_ref[...].reshape(_BLK, _DF)
    hs = jnp.dot(x, wfc_ref[...], preferred_element_type=jnp.float32)
    a1s = jnp.dot(x, wa1_ref[...], preferred_element_type=jnp.float32)
    a2s = jnp.dot(x, wa2_ref[...], preferred_element_type=jnp.float32)
    h = jnp.maximum(hs + bfc_ref[...], 0.0)
    h_ref[...] = h
    a1 = jnp.tanh(a1s + ba1_ref[...])
    a2 = jax.nn.sigmoid(a2s + ba2_ref[...])
    s = jnp.dot(a1 * a2, wa3_ref[...],
                preferred_element_type=jnp.float32) + ba3_ref[0, 0]
    atts_ref[pl.ds(i * _BLK, _BLK), :] = s
    lg = jnp.dot(h, wins_ref[...],
                 preferred_element_type=jnp.float32) + bins_ref[...]
    lg_ref[pl.ds(i * _BLK, _BLK), :] = lg

    m_old = m_ref[0, 0]
    m_new = jnp.maximum(m_old, jnp.max(s))
    scale = jnp.exp(m_old - m_new)
    e = jnp.exp(s - m_new)
    z_ref[0, 0] = z_ref[0, 0] * scale + jnp.sum(e)
    contrib = lax.dot_general(e, h, (((0,), (0,)), ((), ())),
                              preferred_element_type=jnp.float32)
    macc_ref[...] = macc_ref[...] * scale + contrib
    m_ref[0, 0] = m_new

    @pl.when(i == _NB - 1)
    def _fin():
        m = m_ref[0, 0]
        z = z_ref[0, 0]
        atts = atts_ref[...].reshape(_N // 128, 128)
        attlin_ref[...] = atts
        alin_ref[...] = jnp.exp(atts - m) / z
        lg0_ref[...] = lg_ref[:, 0:1].reshape(_N // 128, 128)
        lg1_ref[...] = lg_ref[:, 1:2].reshape(_N // 128, 128)
        big_m = macc_ref[...] / z
        sc = jnp.dot(big_m, wbag_ref[...],
                     preferred_element_type=jnp.float32) + bbag_ref[...]
        score_ref[...] = sc
        ee = jnp.exp(sc - jnp.max(sc))
        p = ee / jnp.sum(ee)
        prob_ref[...] = p
        yhat_ref[...] = jnp.where(sc[0, 1] > sc[0, 0], 1, 0).astype(
            jnp.int32).reshape(1, 1)
        pred_ref[...] = jnp.where(p[0, 1] > p[0, 0], 1, 0).astype(
            jnp.int32).reshape(1, 1)


def _tc_call(x, wfc, bfc, wa1, ba1, wa2, ba2, wa3, ba3, wbag, bbag,
             wins, bins):
    f32 = jnp.float32
    full = lambda shape: pl.BlockSpec(shape, lambda i: (0, 0))
    return pl.pallas_call(
        _tc_body,
        grid=(_NB,),
        in_specs=[
            pl.BlockSpec((_BLK, 8, 128), lambda i: (i, 0, 0)),
            full((_DF, _DC)), full((1, _DC)),
            full((_DF, _DH)), full((1, _DH)),
            full((_DF, _DH)), full((1, _DH)),
            full((_DH, 1)), full((1, 1)),
            full((_DC, _NCLS)), full((1, _NCLS)),
            full((_DC, _NCLS)), full((1, _NCLS)),
        ],
        out_specs=[
            pl.BlockSpec((_BLK, _DC), lambda i: (i, 0)),
            full((_N // 128, 128)),
            full((_N // 128, 128)),
            full((_N // 128, 128)),
            full((_N // 128, 128)),
            full((1, _NCLS)), full((1, _NCLS)),
            full((1, 1)), full((1, 1)),
        ],
        out_shape=[
            jax.ShapeDtypeStruct((_N, _DC), f32),
            jax.ShapeDtypeStruct((_N // 128, 128), f32),
            jax.ShapeDtypeStruct((_N // 128, 128), f32),
            jax.ShapeDtypeStruct((_N // 128, 128), f32),
            jax.ShapeDtypeStruct((_N // 128, 128), f32),
            jax.ShapeDtypeStruct((1, _NCLS), f32),
            jax.ShapeDtypeStruct((1, _NCLS), f32),
            jax.ShapeDtypeStruct((1, 1), jnp.int32),
            jax.ShapeDtypeStruct((1, 1), jnp.int32),
        ],
        scratch_shapes=[
            pltpu.VMEM((_N, 1), f32),
            pltpu.VMEM((1, _DC), f32),
            pltpu.VMEM((_N, _NCLS), f32),
            pltpu.SMEM((1, 1), f32),
            pltpu.SMEM((1, 1), f32),
        ],
        compiler_params=pltpu.CompilerParams(
            dimension_semantics=("arbitrary",)),
    )(x, wfc, bfc, wa1, ba1, wa2, ba2, wa3, ba3, wbag, bbag, wins, bins)


_NW = 16
_CH = _N // _NW
_NCH = _CH // 16


def _select_pass(score_ref, ids_base, n_vregs, is_top, lane_off, k,
                 tvals, tids):
    lanes = lax.iota(jnp.int32, 16)
    fill = _NEG if is_top else _POS

    def body(t, carry):
        tvals, tids = carry
        vs = [score_ref[pl.ds(j * 16, 16)] for j in range(n_vregs)]
        m0 = vs[0]
        for j in range(1, n_vregs):
            m0 = jnp.maximum(m0, vs[j]) if is_top else jnp.minimum(m0, vs[j])
        best = jnp.max(m0) if is_top else jnp.min(m0)

        mi = jnp.full((16,), _IMAX, jnp.int32)
        for j in range(n_vregs):
            idv = lanes + ids_base + j * 16
            mi = jnp.minimum(mi, jnp.where(vs[j] == best, idv, _IMAX))
        chosen = jnp.min(mi)

        tvals = jnp.where(lanes == (t + lane_off), best, tvals)
        tids = jnp.where(lanes == (t + lane_off), chosen, tids)

        for j in range(n_vregs):
            idv = lanes + ids_base + j * 16
            score_ref[pl.ds(j * 16, 16)] = jnp.where(idv == chosen, fill,
                                                     vs[j])
        return (tvals, tids)

    return lax.fori_loop(0, k, body, (tvals, tids))


def _merge_pass(work_ref, cid_ref, is_top, lane_off, gvals, gids):
    lanes = lax.iota(jnp.int32, 16)
    fill = _NEG if is_top else _POS

    def body(t, carry):
        gvals, gids = carry
        vs = [work_ref[pl.ds(j * 16, 16)] for j in range(_NW)]
        ivs = [cid_ref[pl.ds(j * 16, 16)] for j in range(_NW)]
        m0 = vs[0]
        for j in range(1, _NW):
            m0 = jnp.maximum(m0, vs[j]) if is_top else jnp.minimum(m0, vs[j])
        best = jnp.max(m0) if is_top else jnp.min(m0)

        mi = jnp.full((16,), _IMAX, jnp.int32)
        for j in range(_NW):
            mi = jnp.minimum(mi, jnp.where(vs[j] == best, ivs[j], _IMAX))
        chosen = jnp.min(mi)

        gvals = jnp.where(lanes == (t + lane_off), best, gvals)
        gids = jnp.where(lanes == (t + lane_off), chosen, gids)

        for j in range(_NW):
            work_ref[pl.ds(j * 16, 16)] = jnp.where(ivs[j] == chosen, fill,
                                                    vs[j])
        return (gvals, gids)

    return lax.fori_loop(0, _K, body, (gvals, gids))


def _sc_body(att_hbm, l0_hbm, l1_hbm, stagev_hbm, stagei_hbm,
             u0_hbm, u1_hbm, p0_hbm, p1_hbm,
             loc_ref, loc2_ref, sv_ref, si_ref,
             cv_ref, ci_ref, work_ref, cid_ref, l0v_ref, l1v_ref):
    c = lax.axis_index("c")
    s = lax.axis_index("s")
    lanes = lax.iota(jnp.int32, 16)
    zf = jnp.zeros((16,), jnp.float32)
    zi = jnp.zeros((16,), jnp.int32)

    @pl.when(c == 0)
    def _phase1():
        base = s * _CH
        pltpu.sync_copy(att_hbm.at[pl.ds(base, _CH)], loc_ref)
        for j in range(_NCH):
            loc2_ref[pl.ds(j * 16, 16)] = loc_ref[pl.ds(j * 16, 16)]
        tvals, tids = _select_pass(loc_ref, base, _NCH, True, 0, _K, zf, zi)
        tvals, tids = _select_pass(loc2_ref, base, _NCH, False, _K, _K,
                                   tvals, tids)
        sv_ref[...] = tvals
        si_ref[...] = tids
        pltpu.sync_copy(sv_ref, stagev_hbm.at[pl.ds(s * 16, 16)])
        pltpu.sync_copy(si_ref, stagei_hbm.at[pl.ds(s * 16, 16)])

    plsc.subcore_barrier()

    @pl.when(jnp.logical_and(c == 0, s == 0))
    def _merge():
        pltpu.sync_copy(stagev_hbm, cv_ref)
        pltpu.sync_copy(stagei_hbm, ci_ref)

        for j in range(_NW):
            cv = cv_ref[pl.ds(j * 16, 16)]
            ci = ci_ref[pl.ds(j * 16, 16)]
            work_ref[pl.ds(j * 16, 16)] = jnp.where(lanes < _K, cv, _NEG)
            cid_ref[pl.ds(j * 16, 16)] = jnp.where(lanes < _K, ci, _IMAX)
        gvals, gids = _merge_pass(work_ref, cid_ref, True, 0, zf, zi)

        for j in range(_NW):
            cv = cv_ref[pl.ds(j * 16, 16)]
            ci = ci_ref[pl.ds(j * 16, 16)]
            work_ref[pl.ds(j * 16, 16)] = jnp.where(lanes >= _K, cv, _POS)
            cid_ref[pl.ds(j * 16, 16)] = jnp.where(lanes >= _K, ci, _IMAX)
        gvals, gids = _merge_pass(work_ref, cid_ref, False, _K, gvals, gids)

        gids = jnp.clip(gids, 0, _N - 1)

        pltpu.sync_copy(l0_hbm, l0v_ref)
        pltpu.sync_copy(l1_hbm, l1v_ref)
        g0 = plsc.load_gather(l0v_ref, [gids])
        g1 = plsc.load_gather(l1v_ref, [gids])
        mx = jnp.maximum(g0, g1)
        e0 = jnp.exp(g0 - mx)
        e1 = jnp.exp(g1 - mx)
        tot = e0 + e1
        sv_ref[...] = g0
        pltpu.sync_copy(sv_ref, u0_hbm)
        sv_ref[...] = g1
        pltpu.sync_copy(sv_ref, u1_hbm)
        sv_ref[...] = e0 / tot
        pltpu.sync_copy(sv_ref, p0_hbm)
        sv_ref[...] = e1 / tot
        pltpu.sync_copy(sv_ref, p1_hbm)


def _sc_call(att, l0, l1):
    f32 = jnp.float32
    i32 = jnp.int32
    mesh = plsc.VectorSubcoreMesh(core_axis_name="c", subcore_axis_name="s")
    fn = pl.kernel(
        _sc_body,
        out_type=[
            jax.ShapeDtypeStruct((_NW * 16,), f32),
            jax.ShapeDtypeStruct((_NW * 16,), i32),
            jax.ShapeDtypeStruct((16,), f32),
            jax.ShapeDtypeStruct((16,), f32),
            jax.ShapeDtypeStruct((16,), f32),
            jax.ShapeDtypeStruct((16,), f32),
        ],
        mesh=mesh,
        scratch_types=[
            pltpu.VMEM((_CH,), f32),
            pltpu.VMEM((_CH,), f32),
            pltpu.VMEM((16,), f32),
            pltpu.VMEM((16,), i32),
            pltpu.VMEM((_NW * 16,), f32),
            pltpu.VMEM((_NW * 16,), i32),
            pltpu.VMEM((_NW * 16,), f32),
            pltpu.VMEM((_NW * 16,), i32),
            pltpu.VMEM((_N,), f32),
            pltpu.VMEM((_N,), f32),
        ],
        compiler_params=pltpu.CompilerParams(needs_layout_passes=False),
    )
    _, _, u0, u1, p0, p1 = fn(att, l0, l1)
    return u0, u1, p0, p1


def kernel(img_features, slide_label, W_fc, b_fc, W_a1, b_a1, W_a2, b_a2,
           W_a3, b_a3, W_ins, b_ins, W_bag, b_bag):
    x = img_features.reshape(_N, 8, 128)
    (h, attlin, alin, lg0, lg1, score, prob, yhat, pred) = _tc_call(
        x, W_fc, b_fc.reshape(1, _DC), W_a1, b_a1.reshape(1, _DH),
        W_a2, b_a2.reshape(1, _DH), W_a3, b_a3.reshape(1, 1),
        W_bag, b_bag.reshape(1, _NCLS), W_ins, b_ins.reshape(1, _NCLS))

    u0, u1, p0, p1 = _sc_call(attlin.reshape(_N), lg0.reshape(_N),
                              lg1.reshape(_N))
    unnorm = jnp.stack([u0, u1], axis=1)
    ins_prob = jnp.stack([p0, p1], axis=1)

    ins_labels = jnp.concatenate(
        [jnp.ones((_K,), jnp.int32), jnp.zeros((_K,), jnp.int32)], axis=0)
    y_true = jax.nn.one_hot(jnp.asarray(slide_label), _NCLS)
    return (attlin.reshape(_N, 1), alin.reshape(_N, 1), h, ins_labels,
            unnorm, ins_prob, score, prob,
            yhat.reshape(1), y_true, pred.reshape(1))
